# baseline (speedup 1.0000x reference)
import functools

import jax
import jax.numpy as jnp
from jax import lax
from jax.experimental import pallas as pl
from jax.experimental.pallas import tpu as pltpu
from jax.experimental.pallas import tpu_sc as plsc

B, S, H, F, E, K = 2, 2048, 1024, 4096, 8, 2
N = B * S
NPAIR = N * K
M = 512
P = NPAIR + E * M
NBLK = P // M
FB = 512
NJ = F // FB
TB = 512
NTB = N // TB


def _router_body(lg_ref, i1_ref, i2_ref, w1_ref, w2_ref, cnt_ref, ps_ref,
                 loss_ref):
    i = pl.program_id(0)
    lg = lg_ref[...]
    m = jnp.max(lg, axis=1, keepdims=True)
    ex = jnp.exp(lg - m)
    p = ex / jnp.sum(ex, axis=1, keepdims=True)
    cols = jax.lax.broadcasted_iota(jnp.int32, (TB, E), 1)
    p1 = jnp.max(p, axis=1)
    i1 = jnp.min(jnp.where(p == p1[:, None], cols, E), axis=1)
    pm = jnp.where(cols == i1[:, None], -1.0, p)
    p2 = jnp.max(pm, axis=1)
    i2 = jnp.min(jnp.where(pm == p2[:, None], cols, E), axis=1)
    s = p1 + p2 + 1e-9
    i1_ref[...] = i1
    i2_ref[...] = i2
    w1_ref[...] = p1 / s
    w2_ref[...] = p2 / s

    @pl.when(i == 0)
    def _():
        cnt_ref[...] = jnp.zeros_like(cnt_ref)
        ps_ref[...] = jnp.zeros_like(ps_ref)

    onehot = (cols == i1[:, None]).astype(jnp.float32)
    cnt_ref[...] += jnp.sum(onehot, axis=0)[None, :]
    ps_ref[...] += jnp.sum(p, axis=0)[None, :]

    @pl.when(i == NTB - 1)
    def _():
        frac = cnt_ref[...] / float(N)
        avgp = ps_ref[...] / float(N)
        loss_ref[...] = (float(E) * jnp.sum(frac * avgp)).reshape(1, 1)


def _router(logits):
    return pl.pallas_call(
        _router_body,
        grid=(NTB,),
        in_specs=[pl.BlockSpec((TB, E), lambda i: (i, 0))],
        out_specs=[
            pl.BlockSpec((TB,), lambda i: (i,)),
            pl.BlockSpec((TB,), lambda i: (i,)),
            pl.BlockSpec((TB,), lambda i: (i,)),
            pl.BlockSpec((TB,), lambda i: (i,)),
            pl.BlockSpec((1, E), lambda i: (0, 0)),
            pl.BlockSpec((1, E), lambda i: (0, 0)),
            pl.BlockSpec((1, 1), lambda i: (0, 0)),
        ],
        out_shape=[
            jax.ShapeDtypeStruct((N,), jnp.int32),
            jax.ShapeDtypeStruct((N,), jnp.int32),
            jax.ShapeDtypeStruct((N,), jnp.float32),
            jax.ShapeDtypeStruct((N,), jnp.float32),
            jax.ShapeDtypeStruct((1, E), jnp.float32),
            jax.ShapeDtypeStruct((1, E), jnp.float32),
            jax.ShapeDtypeStruct((1, 1), jnp.float32),
        ],
    )(logits)


def _ffn_body(be_ref, act_ref, xs_ref, w1_ref, b1_ref, w2_ref, b2_ref, ws_ref,
              out_ref, acc_ref):
    i = pl.program_id(0)
    j = pl.program_id(1)
    active = act_ref[i] != 0

    @pl.when(active)
    def _():
        x = xs_ref[...]
        h = jnp.dot(x, w1_ref[0], preferred_element_type=jnp.float32)
        h = h + b1_ref[0]
        h = 0.5 * h * (1.0 + jax.lax.erf(h * 0.7071067811865476))
        part = jnp.dot(h, w2_ref[0], preferred_element_type=jnp.float32)

        @pl.when(j == 0)
        def _():
            acc_ref[...] = part

        @pl.when(j > 0)
        def _():
            acc_ref[...] += part

        @pl.when(j == NJ - 1)
        def _():
            out_ref[...] = (acc_ref[...] + b2_ref[0]) * ws_ref[...][:, None]

    @pl.when(jnp.logical_and(jnp.logical_not(active), j == NJ - 1))
    def _():
        out_ref[...] = jnp.zeros_like(out_ref)


def _ffn(xs, w1, b1, w2, b2, ws, block_expert, block_active):
    grid_spec = pltpu.PrefetchScalarGridSpec(
        num_scalar_prefetch=2,
        grid=(NBLK, NJ),
        in_specs=[
            pl.BlockSpec((M, H), lambda i, j, be, act: (i, 0)),
            pl.BlockSpec((1, H, FB), lambda i, j, be, act: (be[i], 0, j)),
            pl.BlockSpec((1, 1, FB), lambda i, j, be, act: (be[i], 0, j)),
            pl.BlockSpec((1, FB, H), lambda i, j, be, act: (be[i], j, 0)),
            pl.BlockSpec((1, 1, H), lambda i, j, be, act: (be[i], 0, 0)),
            pl.BlockSpec((M,), lambda i, j, be, act: (i,)),
        ],
        out_specs=pl.BlockSpec((M, H), lambda i, j, be, act: (i, 0)),
        scratch_shapes=[pltpu.VMEM((M, H), jnp.float32)],
    )
    return pl.pallas_call(
        _ffn_body,
        grid_spec=grid_spec,
        out_shape=jax.ShapeDtypeStruct((P, H), jnp.float32),
    )(block_expert, block_active, xs, w1, b1.reshape(E, 1, F), w2,
      b2.reshape(E, 1, H), ws)


NC, NS, LANES = 2, 16, 16
NW = NC * NS
GCH = 32
CCH = 32
_SC_MESH = dict(core_axis_name="c", subcore_axis_name="s")


def _sc_gather(x, idx, nrows, nslot):
    rows_per_w = nrows // NW
    nch = rows_per_w // GCH
    ngrp = nch // nslot
    assert ngrp * nslot == nch
    NSLOT = nslot

    @functools.partial(
        pl.kernel,
        mesh=plsc.VectorSubcoreMesh(**_SC_MESH),
        out_type=jax.ShapeDtypeStruct((nrows, H), jnp.float32),
        scratch_types=(
            [pltpu.VMEM((NSLOT, GCH), jnp.int32)]
            + [pltpu.VMEM((GCH, H), jnp.float32)] * NSLOT
            + [pltpu.SemaphoreType.DMA] * (2 * NSLOT)
        ),
    )
    def k(x_hbm, idx_hbm, out_hbm, idx_v, *bufs):
        wid = lax.axis_index("s") * NC + lax.axis_index("c")
        base = wid * rows_per_w
        rows = bufs[:NSLOT]
        gsem = bufs[NSLOT:2 * NSLOT]
        ssem = bufs[2 * NSLOT:]

        def start(ch, b):
            pltpu.sync_copy(idx_hbm.at[pl.ds(base + ch * GCH, GCH)],
                            idx_v.at[b])
            pltpu.async_copy(x_hbm.at[idx_v.at[b]], rows[b], gsem[b])

        for b in range(NSLOT):
            start(b, b)

        def group(c2, carry):
            for b in range(NSLOT):
                ch = c2 * NSLOT + b
                pltpu.make_async_copy(x_hbm.at[idx_v.at[b]], rows[b],
                                      gsem[b]).wait()
                pltpu.async_copy(rows[b],
                                 out_hbm.at[pl.ds(base + ch * GCH, GCH)],
                                 ssem[b])

                @pl.when(c2 < ngrp - 1)
                def _():
                    pltpu.make_async_copy(
                        rows[b], out_hbm.at[pl.ds(base, GCH)], ssem[b]).wait()
                    start(ch + NSLOT, b)
            return carry

        lax.fori_loop(0, ngrp, group, 0)
        for b in range(NSLOT):
            pltpu.make_async_copy(rows[b], out_hbm.at[pl.ds(base, GCH)],
                                  ssem[b]).wait()

    return k(x, idx)


def _add_body(a_ref, b_ref, o_ref):
    o_ref[...] = a_ref[...] + b_ref[...]


def _combine_add(ys):
    return pl.pallas_call(
        _add_body,
        grid=(NTB,),
        in_specs=[
            pl.BlockSpec((TB, H), lambda i: (i, 0)),
            pl.BlockSpec((TB, H), lambda i: (i + NTB, 0)),
        ],
        out_specs=pl.BlockSpec((TB, H), lambda i: (i, 0)),
        out_shape=jax.ShapeDtypeStruct((N, H), jnp.float32),
    )(ys, ys)


def kernel(hidden_states, Wr, br, W1, b1, W2, b2):
    x = hidden_states.reshape(-1, H)
    router_logits = x @ Wr.T + br

    i1, i2, w1t, w2t, _cnt, _ps, loss = _router(router_logits)

    e_pairs = jnp.stack([i1, i2], axis=1).reshape(-1)
    onehot = (e_pairs[:, None] == jnp.arange(E)[None, :]).astype(jnp.int32)
    g = jnp.sum(onehot, axis=0)
    rank = jnp.sum(jnp.cumsum(onehot, axis=0) * onehot, axis=1) - 1
    pg = ((g + M - 1) // M) * M
    pstart = jnp.concatenate([jnp.zeros((1,), jnp.int32),
                              jnp.cumsum(pg)[:-1].astype(jnp.int32)])
    pos = pstart[e_pairs] + rank
    tok_for_pos = jnp.zeros((P,), jnp.int32).at[pos].set(
        jnp.arange(NPAIR, dtype=jnp.int32) // K)
    w_pairs = jnp.stack([w1t, w2t], axis=1).reshape(-1)
    w_sorted = jnp.zeros((P,), jnp.float32).at[pos].set(w_pairs)
    inv = pos.reshape(N, K)
    cum_pg = jnp.cumsum(pg)
    block_starts = jnp.arange(NBLK, dtype=jnp.int32) * M
    block_expert = jnp.minimum(
        jnp.sum(block_starts[:, None] >= cum_pg[None, :], axis=1), E - 1
    ).astype(jnp.int32)
    block_active = (
        block_starts < (pstart[block_expert] + g[block_expert])
    ).astype(jnp.int32)

    xs = _sc_gather(x, tok_for_pos, P, 3)
    yw = _ffn(xs, W1, b1, W2, b2, w_sorted, block_expert, block_active)
    idx_all = jnp.concatenate([inv[:, 0], inv[:, 1]])
    ys = _sc_gather(yw, idx_all, 2 * N, 2)
    out = _combine_add(ys)
    return out.reshape(B, S, H), loss.reshape(())


# speedup vs baseline: 1.3804x; 1.3804x over previous
import functools

import jax
import jax.numpy as jnp
from jax import lax
from jax.experimental import pallas as pl
from jax.experimental.pallas import tpu as pltpu
from jax.experimental.pallas import tpu_sc as plsc

B, S, H, F, E, K = 2, 2048, 1024, 4096, 8, 2
N = B * S
NPAIR = N * K
M = 512
P = NPAIR + E * M
NBLK = P // M
FB = 512
NJ = F // FB
TB = 512
NTB = N // TB


def _router_body(lg_ref, i1_ref, i2_ref, w1_ref, w2_ref, cnt_ref, ps_ref,
                 loss_ref):
    i = pl.program_id(0)
    lg = lg_ref[...]
    m = jnp.max(lg, axis=1, keepdims=True)
    ex = jnp.exp(lg - m)
    p = ex / jnp.sum(ex, axis=1, keepdims=True)
    cols = jax.lax.broadcasted_iota(jnp.int32, (TB, E), 1)
    p1 = jnp.max(p, axis=1)
    i1 = jnp.min(jnp.where(p == p1[:, None], cols, E), axis=1)
    pm = jnp.where(cols == i1[:, None], -1.0, p)
    p2 = jnp.max(pm, axis=1)
    i2 = jnp.min(jnp.where(pm == p2[:, None], cols, E), axis=1)
    s = p1 + p2 + 1e-9
    i1_ref[...] = i1
    i2_ref[...] = i2
    w1_ref[...] = p1 / s
    w2_ref[...] = p2 / s

    @pl.when(i == 0)
    def _():
        cnt_ref[...] = jnp.zeros_like(cnt_ref)
        ps_ref[...] = jnp.zeros_like(ps_ref)

    onehot = (cols == i1[:, None]).astype(jnp.float32)
    cnt_ref[...] += jnp.sum(onehot, axis=0)[None, :]
    ps_ref[...] += jnp.sum(p, axis=0)[None, :]

    @pl.when(i == NTB - 1)
    def _():
        frac = cnt_ref[...] / float(N)
        avgp = ps_ref[...] / float(N)
        loss_ref[...] = (float(E) * jnp.sum(frac * avgp)).reshape(1, 1)


def _router(logits):
    return pl.pallas_call(
        _router_body,
        grid=(NTB,),
        in_specs=[pl.BlockSpec((TB, E), lambda i: (i, 0))],
        out_specs=[
            pl.BlockSpec((TB,), lambda i: (i,)),
            pl.BlockSpec((TB,), lambda i: (i,)),
            pl.BlockSpec((TB,), lambda i: (i,)),
            pl.BlockSpec((TB,), lambda i: (i,)),
            pl.BlockSpec((1, E), lambda i: (0, 0)),
            pl.BlockSpec((1, E), lambda i: (0, 0)),
            pl.BlockSpec((1, 1), lambda i: (0, 0)),
        ],
        out_shape=[
            jax.ShapeDtypeStruct((N,), jnp.int32),
            jax.ShapeDtypeStruct((N,), jnp.int32),
            jax.ShapeDtypeStruct((N,), jnp.float32),
            jax.ShapeDtypeStruct((N,), jnp.float32),
            jax.ShapeDtypeStruct((1, E), jnp.float32),
            jax.ShapeDtypeStruct((1, E), jnp.float32),
            jax.ShapeDtypeStruct((1, 1), jnp.float32),
        ],
    )(logits)


def _ffn_body(be_ref, act_ref, xs_ref, w1_ref, b1_ref, w2_ref, b2_ref, ws_ref,
              out_ref, acc_ref):
    i = pl.program_id(0)
    j = pl.program_id(1)
    active = act_ref[i] != 0

    @pl.when(active)
    def _():
        x = xs_ref[...]
        h = jnp.dot(x, w1_ref[0], preferred_element_type=jnp.float32)
        h = h + b1_ref[0]
        h = 0.5 * h * (1.0 + jax.lax.erf(h * 0.7071067811865476))
        part = jnp.dot(h, w2_ref[0], preferred_element_type=jnp.float32)

        @pl.when(j == 0)
        def _():
            acc_ref[...] = part

        @pl.when(j > 0)
        def _():
            acc_ref[...] += part

        @pl.when(j == NJ - 1)
        def _():
            out_ref[...] = (acc_ref[...] + b2_ref[0]) * ws_ref[...][:, None]

    @pl.when(jnp.logical_and(jnp.logical_not(active), j == NJ - 1))
    def _():
        out_ref[...] = jnp.zeros_like(out_ref)


def _ffn(xs, w1, b1, w2, b2, ws, block_expert, block_active):
    grid_spec = pltpu.PrefetchScalarGridSpec(
        num_scalar_prefetch=2,
        grid=(NBLK, NJ),
        in_specs=[
            pl.BlockSpec((M, H), lambda i, j, be, act: (i, 0)),
            pl.BlockSpec((1, H, FB), lambda i, j, be, act: (be[i], 0, j)),
            pl.BlockSpec((1, 1, FB), lambda i, j, be, act: (be[i], 0, j)),
            pl.BlockSpec((1, FB, H), lambda i, j, be, act: (be[i], j, 0)),
            pl.BlockSpec((1, 1, H), lambda i, j, be, act: (be[i], 0, 0)),
            pl.BlockSpec((M,), lambda i, j, be, act: (i,)),
        ],
        out_specs=pl.BlockSpec((M, H), lambda i, j, be, act: (i, 0)),
        scratch_shapes=[pltpu.VMEM((M, H), jnp.float32)],
    )
    return pl.pallas_call(
        _ffn_body,
        grid_spec=grid_spec,
        out_shape=jax.ShapeDtypeStruct((P, H), jnp.float32),
    )(block_expert, block_active, xs, w1, b1.reshape(E, 1, F), w2,
      b2.reshape(E, 1, H), ws)


NC, NS, LANES = 2, 16, 16
NW = NC * NS
GCH = 32
CCH = 32
_SC_MESH = dict(core_axis_name="c", subcore_axis_name="s")


def _sc_gather(x, idx, nrows, nslot):
    rows_per_w = nrows // NW
    nch = rows_per_w // GCH
    ngrp = nch // nslot
    assert ngrp * nslot == nch
    NSLOT = nslot

    @functools.partial(
        pl.kernel,
        mesh=plsc.VectorSubcoreMesh(**_SC_MESH),
        out_type=jax.ShapeDtypeStruct((nrows, H), jnp.float32),
        scratch_types=(
            [pltpu.VMEM((NSLOT, GCH), jnp.int32)]
            + [pltpu.VMEM((GCH, H), jnp.float32)] * NSLOT
            + [pltpu.SemaphoreType.DMA] * (2 * NSLOT)
        ),
    )
    def k(x_hbm, idx_hbm, out_hbm, idx_v, *bufs):
        wid = lax.axis_index("s") * NC + lax.axis_index("c")
        base = wid * rows_per_w
        rows = bufs[:NSLOT]
        gsem = bufs[NSLOT:2 * NSLOT]
        ssem = bufs[2 * NSLOT:]

        def start(ch, b):
            pltpu.sync_copy(idx_hbm.at[pl.ds(base + ch * GCH, GCH)],
                            idx_v.at[b])
            pltpu.async_copy(x_hbm.at[idx_v.at[b]], rows[b], gsem[b])

        for b in range(NSLOT):
            start(b, b)

        def group(c2, carry):
            for b in range(NSLOT):
                ch = c2 * NSLOT + b
                pltpu.make_async_copy(x_hbm.at[idx_v.at[b]], rows[b],
                                      gsem[b]).wait()
                pltpu.async_copy(rows[b],
                                 out_hbm.at[pl.ds(base + ch * GCH, GCH)],
                                 ssem[b])

                @pl.when(c2 < ngrp - 1)
                def _():
                    pltpu.make_async_copy(
                        rows[b], out_hbm.at[pl.ds(base, GCH)], ssem[b]).wait()
                    start(ch + NSLOT, b)
            return carry

        lax.fori_loop(0, ngrp, group, 0)
        for b in range(NSLOT):
            pltpu.make_async_copy(rows[b], out_hbm.at[pl.ds(base, GCH)],
                                  ssem[b]).wait()

    return k(x, idx)


def _add_body(a_ref, b_ref, o_ref):
    o_ref[...] = a_ref[...] + b_ref[...]


def _combine_add(ys):
    return pl.pallas_call(
        _add_body,
        grid=(NTB,),
        in_specs=[
            pl.BlockSpec((TB, H), lambda i: (i, 0)),
            pl.BlockSpec((TB, H), lambda i: (i + NTB, 0)),
        ],
        out_specs=pl.BlockSpec((TB, H), lambda i: (i, 0)),
        out_shape=jax.ShapeDtypeStruct((N, H), jnp.float32),
    )(ys, ys)


def kernel(hidden_states, Wr, br, W1, b1, W2, b2):
    x = hidden_states.reshape(-1, H)
    router_logits = x @ Wr.T + br

    i1, i2, w1t, w2t, _cnt, _ps, loss = _router(router_logits)

    e_pairs = jnp.stack([i1, i2], axis=1).reshape(-1)
    onehot = (e_pairs[:, None] == jnp.arange(E)[None, :]).astype(jnp.int32)
    g = jnp.sum(onehot, axis=0)
    rank = jnp.sum(jnp.cumsum(onehot, axis=0) * onehot, axis=1) - 1
    pg = ((g + M - 1) // M) * M
    pstart = jnp.concatenate([jnp.zeros((1,), jnp.int32),
                              jnp.cumsum(pg)[:-1].astype(jnp.int32)])
    pos = pstart[e_pairs] + rank
    tok_for_pos = (jnp.arange(P, dtype=jnp.int32) % N).at[pos].set(
        jnp.arange(NPAIR, dtype=jnp.int32) // K)
    w_pairs = jnp.stack([w1t, w2t], axis=1).reshape(-1)
    w_sorted = jnp.zeros((P,), jnp.float32).at[pos].set(w_pairs)
    inv = pos.reshape(N, K)
    cum_pg = jnp.cumsum(pg)
    block_starts = jnp.arange(NBLK, dtype=jnp.int32) * M
    block_expert = jnp.minimum(
        jnp.sum(block_starts[:, None] >= cum_pg[None, :], axis=1), E - 1
    ).astype(jnp.int32)
    block_active = (
        block_starts < (pstart[block_expert] + g[block_expert])
    ).astype(jnp.int32)

    xs = _sc_gather(x, tok_for_pos, P, 3)
    yw = _ffn(xs, W1, b1, W2, b2, w_sorted, block_expert, block_active)
    idx_all = jnp.concatenate([inv[:, 0], inv[:, 1]])
    ys = _sc_gather(yw, idx_all, 2 * N, 2)
    out = _combine_add(ys)
    return out.reshape(B, S, H), loss.reshape(())


# speedup vs baseline: 1.5519x; 1.1242x over previous
import functools

import jax
import jax.numpy as jnp
from jax import lax
from jax.experimental import pallas as pl
from jax.experimental.pallas import tpu as pltpu
from jax.experimental.pallas import tpu_sc as plsc

B, S, H, F, E, K = 2, 2048, 1024, 4096, 8, 2
N = B * S
NPAIR = N * K
M = 512
P = NPAIR + E * M
NBLK = P // M
FB = 1024
NJ = F // FB
TB = 512
NTB = N // TB


def _router_body(lg_ref, i1_ref, i2_ref, w1_ref, w2_ref, cnt_ref, ps_ref,
                 loss_ref):
    i = pl.program_id(0)
    lg = lg_ref[...]
    m = jnp.max(lg, axis=1, keepdims=True)
    ex = jnp.exp(lg - m)
    p = ex / jnp.sum(ex, axis=1, keepdims=True)
    cols = jax.lax.broadcasted_iota(jnp.int32, (TB, E), 1)
    p1 = jnp.max(p, axis=1)
    i1 = jnp.min(jnp.where(p == p1[:, None], cols, E), axis=1)
    pm = jnp.where(cols == i1[:, None], -1.0, p)
    p2 = jnp.max(pm, axis=1)
    i2 = jnp.min(jnp.where(pm == p2[:, None], cols, E), axis=1)
    s = p1 + p2 + 1e-9
    i1_ref[...] = i1
    i2_ref[...] = i2
    w1_ref[...] = p1 / s
    w2_ref[...] = p2 / s

    @pl.when(i == 0)
    def _():
        cnt_ref[...] = jnp.zeros_like(cnt_ref)
        ps_ref[...] = jnp.zeros_like(ps_ref)

    onehot = (cols == i1[:, None]).astype(jnp.float32)
    cnt_ref[...] += jnp.sum(onehot, axis=0)[None, :]
    ps_ref[...] += jnp.sum(p, axis=0)[None, :]

    @pl.when(i == NTB - 1)
    def _():
        frac = cnt_ref[...] / float(N)
        avgp = ps_ref[...] / float(N)
        loss_ref[...] = (float(E) * jnp.sum(frac * avgp)).reshape(1, 1)


def _router(logits):
    return pl.pallas_call(
        _router_body,
        grid=(NTB,),
        in_specs=[pl.BlockSpec((TB, E), lambda i: (i, 0))],
        out_specs=[
            pl.BlockSpec((TB,), lambda i: (i,)),
            pl.BlockSpec((TB,), lambda i: (i,)),
            pl.BlockSpec((TB,), lambda i: (i,)),
            pl.BlockSpec((TB,), lambda i: (i,)),
            pl.BlockSpec((1, E), lambda i: (0, 0)),
            pl.BlockSpec((1, E), lambda i: (0, 0)),
            pl.BlockSpec((1, 1), lambda i: (0, 0)),
        ],
        out_shape=[
            jax.ShapeDtypeStruct((N,), jnp.int32),
            jax.ShapeDtypeStruct((N,), jnp.int32),
            jax.ShapeDtypeStruct((N,), jnp.float32),
            jax.ShapeDtypeStruct((N,), jnp.float32),
            jax.ShapeDtypeStruct((1, E), jnp.float32),
            jax.ShapeDtypeStruct((1, E), jnp.float32),
            jax.ShapeDtypeStruct((1, 1), jnp.float32),
        ],
    )(logits)


def _ffn_body(be_ref, act_ref, xs_ref, w1_ref, b1_ref, w2_ref, b2_ref, ws_ref,
              out_ref, acc_ref):
    i = pl.program_id(0)
    j = pl.program_id(1)
    active = act_ref[i] != 0

    @pl.when(active)
    def _():
        x = xs_ref[...]
        h = jnp.dot(x, w1_ref[0], preferred_element_type=jnp.float32)
        h = h + b1_ref[0]
        h = 0.5 * h * (1.0 + jax.lax.erf(h * 0.7071067811865476))
        part = jnp.dot(h, w2_ref[0], preferred_element_type=jnp.float32)

        @pl.when(j == 0)
        def _():
            acc_ref[...] = part

        @pl.when(j > 0)
        def _():
            acc_ref[...] += part

        @pl.when(j == NJ - 1)
        def _():
            out_ref[...] = (acc_ref[...] + b2_ref[0]) * ws_ref[...][:, None]

    @pl.when(jnp.logical_and(jnp.logical_not(active), j == NJ - 1))
    def _():
        out_ref[...] = jnp.zeros_like(out_ref)


def _ffn(xs, w1, b1, w2, b2, ws, block_expert, block_active):
    grid_spec = pltpu.PrefetchScalarGridSpec(
        num_scalar_prefetch=2,
        grid=(NBLK, NJ),
        in_specs=[
            pl.BlockSpec((M, H), lambda i, j, be, act: (i, 0)),
            pl.BlockSpec((1, H, FB), lambda i, j, be, act: (be[i], 0, j)),
            pl.BlockSpec((1, 1, FB), lambda i, j, be, act: (be[i], 0, j)),
            pl.BlockSpec((1, FB, H), lambda i, j, be, act: (be[i], j, 0)),
            pl.BlockSpec((1, 1, H), lambda i, j, be, act: (be[i], 0, 0)),
            pl.BlockSpec((M,), lambda i, j, be, act: (i,)),
        ],
        out_specs=pl.BlockSpec((M, H), lambda i, j, be, act: (i, 0)),
        scratch_shapes=[pltpu.VMEM((M, H), jnp.float32)],
    )
    return pl.pallas_call(
        _ffn_body,
        grid_spec=grid_spec,
        out_shape=jax.ShapeDtypeStruct((P, H), jnp.float32),
    )(block_expert, block_active, xs, w1, b1.reshape(E, 1, F), w2,
      b2.reshape(E, 1, H), ws)


NC, NS, LANES = 2, 16, 16
NW = NC * NS
GCH = 32
CCH = 32
_SC_MESH = dict(core_axis_name="c", subcore_axis_name="s")


def _sc_gather(x, idx, nrows, nslot):
    rows_per_w = nrows // NW
    nch = rows_per_w // GCH
    ngrp = nch // nslot
    assert ngrp * nslot == nch
    NSLOT = nslot

    @functools.partial(
        pl.kernel,
        mesh=plsc.VectorSubcoreMesh(**_SC_MESH),
        out_type=jax.ShapeDtypeStruct((nrows, H), jnp.float32),
        scratch_types=(
            [pltpu.VMEM((NSLOT, GCH), jnp.int32)]
            + [pltpu.VMEM((GCH, H), jnp.float32)] * NSLOT
            + [pltpu.SemaphoreType.DMA] * (2 * NSLOT)
        ),
    )
    def k(x_hbm, idx_hbm, out_hbm, idx_v, *bufs):
        wid = lax.axis_index("s") * NC + lax.axis_index("c")
        base = wid * rows_per_w
        rows = bufs[:NSLOT]
        gsem = bufs[NSLOT:2 * NSLOT]
        ssem = bufs[2 * NSLOT:]

        def start(ch, b):
            pltpu.sync_copy(idx_hbm.at[pl.ds(base + ch * GCH, GCH)],
                            idx_v.at[b])
            pltpu.async_copy(x_hbm.at[idx_v.at[b]], rows[b], gsem[b])

        for b in range(NSLOT):
            start(b, b)

        def group(c2, carry):
            for b in range(NSLOT):
                ch = c2 * NSLOT + b
                pltpu.make_async_copy(x_hbm.at[idx_v.at[b]], rows[b],
                                      gsem[b]).wait()
                pltpu.async_copy(rows[b],
                                 out_hbm.at[pl.ds(base + ch * GCH, GCH)],
                                 ssem[b])

                @pl.when(c2 < ngrp - 1)
                def _():
                    pltpu.make_async_copy(
                        rows[b], out_hbm.at[pl.ds(base, GCH)], ssem[b]).wait()
                    start(ch + NSLOT, b)
            return carry

        lax.fori_loop(0, ngrp, group, 0)
        for b in range(NSLOT):
            pltpu.make_async_copy(rows[b], out_hbm.at[pl.ds(base, GCH)],
                                  ssem[b]).wait()

    return k(x, idx)


def _add_body(a_ref, b_ref, o_ref):
    o_ref[...] = a_ref[...] + b_ref[...]


def _combine_add(ys):
    return pl.pallas_call(
        _add_body,
        grid=(NTB,),
        in_specs=[
            pl.BlockSpec((TB, H), lambda i: (i, 0)),
            pl.BlockSpec((TB, H), lambda i: (i + NTB, 0)),
        ],
        out_specs=pl.BlockSpec((TB, H), lambda i: (i, 0)),
        out_shape=jax.ShapeDtypeStruct((N, H), jnp.float32),
    )(ys, ys)


def kernel(hidden_states, Wr, br, W1, b1, W2, b2):
    x = hidden_states.reshape(-1, H)
    router_logits = x @ Wr.T + br

    i1, i2, w1t, w2t, _cnt, _ps, loss = _router(router_logits)

    e_pairs = jnp.stack([i1, i2], axis=1).reshape(-1)
    onehot = (e_pairs[:, None] == jnp.arange(E)[None, :]).astype(jnp.int32)
    g = jnp.sum(onehot, axis=0)
    rank = jnp.sum(jnp.cumsum(onehot, axis=0) * onehot, axis=1) - 1
    pg = ((g + M - 1) // M) * M
    pstart = jnp.concatenate([jnp.zeros((1,), jnp.int32),
                              jnp.cumsum(pg)[:-1].astype(jnp.int32)])
    pos = pstart[e_pairs] + rank
    tok_for_pos = (jnp.arange(P, dtype=jnp.int32) % N).at[pos].set(
        jnp.arange(NPAIR, dtype=jnp.int32) // K)
    w_pairs = jnp.stack([w1t, w2t], axis=1).reshape(-1)
    w_sorted = jnp.zeros((P,), jnp.float32).at[pos].set(w_pairs)
    inv = pos.reshape(N, K)
    cum_pg = jnp.cumsum(pg)
    block_starts = jnp.arange(NBLK, dtype=jnp.int32) * M
    block_expert = jnp.minimum(
        jnp.sum(block_starts[:, None] >= cum_pg[None, :], axis=1), E - 1
    ).astype(jnp.int32)
    block_active = (
        block_starts < (pstart[block_expert] + g[block_expert])
    ).astype(jnp.int32)

    xs = _sc_gather(x, tok_for_pos, P, 3)
    yw = _ffn(xs, W1, b1, W2, b2, w_sorted, block_expert, block_active)
    idx_all = jnp.concatenate([inv[:, 0], inv[:, 1]])
    ys = _sc_gather(yw, idx_all, 2 * N, 2)
    out = _combine_add(ys)
    return out.reshape(B, S, H), loss.reshape(())


# speedup vs baseline: 1.6560x; 1.0671x over previous
import functools

import jax
import jax.numpy as jnp
from jax import lax
from jax.experimental import pallas as pl
from jax.experimental.pallas import tpu as pltpu
from jax.experimental.pallas import tpu_sc as plsc

B, S, H, F, E, K = 2, 2048, 1024, 4096, 8, 2
N = B * S
NPAIR = N * K
M = 512
P = NPAIR + E * M
NBLK = P // M
FB = 2048
NJ = F // FB
TB = 512
NTB = N // TB


def _router_body(lg_ref, i1_ref, i2_ref, w1_ref, w2_ref, cnt_ref, ps_ref,
                 loss_ref):
    i = pl.program_id(0)
    lg = lg_ref[...]
    m = jnp.max(lg, axis=1, keepdims=True)
    ex = jnp.exp(lg - m)
    p = ex / jnp.sum(ex, axis=1, keepdims=True)
    cols = jax.lax.broadcasted_iota(jnp.int32, (TB, E), 1)
    p1 = jnp.max(p, axis=1)
    i1 = jnp.min(jnp.where(p == p1[:, None], cols, E), axis=1)
    pm = jnp.where(cols == i1[:, None], -1.0, p)
    p2 = jnp.max(pm, axis=1)
    i2 = jnp.min(jnp.where(pm == p2[:, None], cols, E), axis=1)
    s = p1 + p2 + 1e-9
    i1_ref[...] = i1
    i2_ref[...] = i2
    w1_ref[...] = p1 / s
    w2_ref[...] = p2 / s

    @pl.when(i == 0)
    def _():
        cnt_ref[...] = jnp.zeros_like(cnt_ref)
        ps_ref[...] = jnp.zeros_like(ps_ref)

    onehot = (cols == i1[:, None]).astype(jnp.float32)
    cnt_ref[...] += jnp.sum(onehot, axis=0)[None, :]
    ps_ref[...] += jnp.sum(p, axis=0)[None, :]

    @pl.when(i == NTB - 1)
    def _():
        frac = cnt_ref[...] / float(N)
        avgp = ps_ref[...] / float(N)
        loss_ref[...] = (float(E) * jnp.sum(frac * avgp)).reshape(1, 1)


def _router(logits):
    return pl.pallas_call(
        _router_body,
        grid=(NTB,),
        in_specs=[pl.BlockSpec((TB, E), lambda i: (i, 0))],
        out_specs=[
            pl.BlockSpec((TB,), lambda i: (i,)),
            pl.BlockSpec((TB,), lambda i: (i,)),
            pl.BlockSpec((TB,), lambda i: (i,)),
            pl.BlockSpec((TB,), lambda i: (i,)),
            pl.BlockSpec((1, E), lambda i: (0, 0)),
            pl.BlockSpec((1, E), lambda i: (0, 0)),
            pl.BlockSpec((1, 1), lambda i: (0, 0)),
        ],
        out_shape=[
            jax.ShapeDtypeStruct((N,), jnp.int32),
            jax.ShapeDtypeStruct((N,), jnp.int32),
            jax.ShapeDtypeStruct((N,), jnp.float32),
            jax.ShapeDtypeStruct((N,), jnp.float32),
            jax.ShapeDtypeStruct((1, E), jnp.float32),
            jax.ShapeDtypeStruct((1, E), jnp.float32),
            jax.ShapeDtypeStruct((1, 1), jnp.float32),
        ],
    )(logits)


def _ffn_body(be_ref, act_ref, xs_ref, w1_ref, b1_ref, w2_ref, b2_ref, ws_ref,
              out_ref, acc_ref):
    i = pl.program_id(0)
    j = pl.program_id(1)
    active = act_ref[i] != 0

    @pl.when(active)
    def _():
        x = xs_ref[...]
        h = jnp.dot(x, w1_ref[0], preferred_element_type=jnp.float32)
        h = h + b1_ref[0]
        h = 0.5 * h * (1.0 + jax.lax.erf(h * 0.7071067811865476))
        part = jnp.dot(h, w2_ref[0], preferred_element_type=jnp.float32)

        @pl.when(j == 0)
        def _():
            acc_ref[...] = part

        @pl.when(j > 0)
        def _():
            acc_ref[...] += part

        @pl.when(j == NJ - 1)
        def _():
            out_ref[...] = (acc_ref[...] + b2_ref[0]) * ws_ref[...][:, None]

    @pl.when(jnp.logical_and(jnp.logical_not(active), j == NJ - 1))
    def _():
        out_ref[...] = jnp.zeros_like(out_ref)


def _ffn(xs, w1, b1, w2, b2, ws, block_expert, block_active):
    grid_spec = pltpu.PrefetchScalarGridSpec(
        num_scalar_prefetch=2,
        grid=(NBLK, NJ),
        in_specs=[
            pl.BlockSpec((M, H), lambda i, j, be, act: (i, 0)),
            pl.BlockSpec((1, H, FB), lambda i, j, be, act: (be[i], 0, j)),
            pl.BlockSpec((1, 1, FB), lambda i, j, be, act: (be[i], 0, j)),
            pl.BlockSpec((1, FB, H), lambda i, j, be, act: (be[i], j, 0)),
            pl.BlockSpec((1, 1, H), lambda i, j, be, act: (be[i], 0, 0)),
            pl.BlockSpec((M,), lambda i, j, be, act: (i,)),
        ],
        out_specs=pl.BlockSpec((M, H), lambda i, j, be, act: (i, 0)),
        scratch_shapes=[pltpu.VMEM((M, H), jnp.float32)],
    )
    return pl.pallas_call(
        _ffn_body,
        grid_spec=grid_spec,
        out_shape=jax.ShapeDtypeStruct((P, H), jnp.float32),
    )(block_expert, block_active, xs, w1, b1.reshape(E, 1, F), w2,
      b2.reshape(E, 1, H), ws)


NC, NS, LANES = 2, 16, 16
NW = NC * NS
GCH = 32
CCH = 32
_SC_MESH = dict(core_axis_name="c", subcore_axis_name="s")


def _sc_gather(x, idx, nrows, nslot):
    rows_per_w = nrows // NW
    nch = rows_per_w // GCH
    ngrp = nch // nslot
    assert ngrp * nslot == nch
    NSLOT = nslot

    @functools.partial(
        pl.kernel,
        mesh=plsc.VectorSubcoreMesh(**_SC_MESH),
        out_type=jax.ShapeDtypeStruct((nrows, H), jnp.float32),
        scratch_types=(
            [pltpu.VMEM((NSLOT, GCH), jnp.int32)]
            + [pltpu.VMEM((GCH, H), jnp.float32)] * NSLOT
            + [pltpu.SemaphoreType.DMA] * (2 * NSLOT)
        ),
    )
    def k(x_hbm, idx_hbm, out_hbm, idx_v, *bufs):
        wid = lax.axis_index("s") * NC + lax.axis_index("c")
        base = wid * rows_per_w
        rows = bufs[:NSLOT]
        gsem = bufs[NSLOT:2 * NSLOT]
        ssem = bufs[2 * NSLOT:]

        def start(ch, b):
            pltpu.sync_copy(idx_hbm.at[pl.ds(base + ch * GCH, GCH)],
                            idx_v.at[b])
            pltpu.async_copy(x_hbm.at[idx_v.at[b]], rows[b], gsem[b])

        for b in range(NSLOT):
            start(b, b)

        def group(c2, carry):
            for b in range(NSLOT):
                ch = c2 * NSLOT + b
                pltpu.make_async_copy(x_hbm.at[idx_v.at[b]], rows[b],
                                      gsem[b]).wait()
                pltpu.async_copy(rows[b],
                                 out_hbm.at[pl.ds(base + ch * GCH, GCH)],
                                 ssem[b])

                @pl.when(c2 < ngrp - 1)
                def _():
                    pltpu.make_async_copy(
                        rows[b], out_hbm.at[pl.ds(base, GCH)], ssem[b]).wait()
                    start(ch + NSLOT, b)
            return carry

        lax.fori_loop(0, ngrp, group, 0)
        for b in range(NSLOT):
            pltpu.make_async_copy(rows[b], out_hbm.at[pl.ds(base, GCH)],
                                  ssem[b]).wait()

    return k(x, idx)


def _add_body(a_ref, b_ref, o_ref):
    o_ref[...] = a_ref[...] + b_ref[...]


def _combine_add(ys):
    return pl.pallas_call(
        _add_body,
        grid=(NTB,),
        in_specs=[
            pl.BlockSpec((TB, H), lambda i: (i, 0)),
            pl.BlockSpec((TB, H), lambda i: (i + NTB, 0)),
        ],
        out_specs=pl.BlockSpec((TB, H), lambda i: (i, 0)),
        out_shape=jax.ShapeDtypeStruct((N, H), jnp.float32),
    )(ys, ys)


def kernel(hidden_states, Wr, br, W1, b1, W2, b2):
    x = hidden_states.reshape(-1, H)
    router_logits = x @ Wr.T + br

    i1, i2, w1t, w2t, _cnt, _ps, loss = _router(router_logits)

    e_pairs = jnp.stack([i1, i2], axis=1).reshape(-1)
    onehot = (e_pairs[:, None] == jnp.arange(E)[None, :]).astype(jnp.int32)
    g = jnp.sum(onehot, axis=0)
    rank = jnp.sum(jnp.cumsum(onehot, axis=0) * onehot, axis=1) - 1
    pg = ((g + M - 1) // M) * M
    pstart = jnp.concatenate([jnp.zeros((1,), jnp.int32),
                              jnp.cumsum(pg)[:-1].astype(jnp.int32)])
    pos = pstart[e_pairs] + rank
    tok_for_pos = (jnp.arange(P, dtype=jnp.int32) % N).at[pos].set(
        jnp.arange(NPAIR, dtype=jnp.int32) // K)
    w_pairs = jnp.stack([w1t, w2t], axis=1).reshape(-1)
    w_sorted = jnp.zeros((P,), jnp.float32).at[pos].set(w_pairs)
    inv = pos.reshape(N, K)
    cum_pg = jnp.cumsum(pg)
    block_starts = jnp.arange(NBLK, dtype=jnp.int32) * M
    block_expert = jnp.minimum(
        jnp.sum(block_starts[:, None] >= cum_pg[None, :], axis=1), E - 1
    ).astype(jnp.int32)
    block_active = (
        block_starts < (pstart[block_expert] + g[block_expert])
    ).astype(jnp.int32)

    xs = _sc_gather(x, tok_for_pos, P, 3)
    yw = _ffn(xs, W1, b1, W2, b2, w_sorted, block_expert, block_active)
    idx_all = jnp.concatenate([inv[:, 0], inv[:, 1]])
    ys = _sc_gather(yw, idx_all, 2 * N, 2)
    out = _combine_add(ys)
    return out.reshape(B, S, H), loss.reshape(())
